# baseline (speedup 1.0000x reference)
import functools
import math

import jax
import jax.numpy as jnp
from jax import lax
from jax.experimental import pallas as pl
from jax.experimental.pallas import tpu as pltpu
from jax.experimental.pallas import tpu_sc as plsc

VOCAB = 1000000
DIM = 64
BATCH = 1024
SEQ = 200
B = BATCH * SEQ
NW = 32
PER_W = B // NW
CHUNK = 128
NCHUNK = PER_W // CHUNK
ROW = 4 * DIM

_LN10000_OVER_DIM = math.log(10000.0) / DIM


def _rsqrt_nr(s):
    half = s * 0.5
    bits = lax.bitcast_convert_type(s, jnp.int32)
    bits = jnp.int32(0x5F3759DF) - (bits >> 1)
    y = lax.bitcast_convert_type(bits, jnp.float32)
    y = y * (1.5 - half * y * y)
    y = y * (1.5 - half * y * y)
    return y


def _body(idx_hbm, scalar_hbm, vi_hbm, vj_hbm, vk_hbm, out_hbm,
          idx_v, gr, gi, gj, gk, obuf, sem):
    wid = lax.axis_index("s") * 2 + lax.axis_index("c")
    base = wid * PER_W

    pltpu.sync_copy(idx_hbm.at[wid], idx_v)

    lane = lax.iota(jnp.int32, 16)
    lane4 = lane * 4
    lane_f = lane.astype(jnp.float32)
    scales = [jnp.exp((lane_f + (g * 16)) * (-_LN10000_OVER_DIM))
              for g in range(DIM // 16)]

    def chunk_body(t, carry):
        idx_row = idx_v.at[t]
        c0 = pltpu.async_copy(scalar_hbm.at[idx_row], gr, sem)
        c1 = pltpu.async_copy(vi_hbm.at[idx_row], gi, sem)
        c2 = pltpu.async_copy(vj_hbm.at[idx_row], gj, sem)
        c3 = pltpu.async_copy(vk_hbm.at[idx_row], gk, sem)
        c0.wait()
        c1.wait()
        c2.wait()
        c3.wait()

        def n_body(n, carry2):
            nvec = jnp.full((16,), n, dtype=jnp.int32)
            for g in range(DIM // 16):
                sl = pl.ds(g * 16, 16)
                r = gr[n, sl]
                i = gi[n, sl] * scales[g]
                j = gj[n, sl] * scales[g]
                k = gk[n, sl] * scales[g]
                y = _rsqrt_nr(r * r + i * i + j * j + k * k + 1e-6)
                col = lane4 + (g * 64)
                plsc.store_scatter(obuf, [nvec, col], r * y)
                plsc.store_scatter(obuf, [nvec, col + 1], i * y)
                plsc.store_scatter(obuf, [nvec, col + 2], j * y)
                plsc.store_scatter(obuf, [nvec, col + 3], k * y)
            return carry2

        lax.fori_loop(0, CHUNK, n_body, 0)
        pltpu.sync_copy(obuf, out_hbm.at[pl.ds(base + t * CHUNK, CHUNK)])
        return carry

    lax.fori_loop(0, NCHUNK, chunk_body, 0)


def kernel(x, scalar, vector_i, vector_j, vector_k):
    idx = x.reshape(NW, NCHUNK, CHUNK)
    mesh = plsc.VectorSubcoreMesh(core_axis_name="c", subcore_axis_name="s")
    f = functools.partial(
        pl.kernel,
        mesh=mesh,
        compiler_params=pltpu.CompilerParams(use_tc_tiling_on_sc=False,
                                             needs_layout_passes=False),
        out_type=jax.ShapeDtypeStruct((B, ROW), jnp.float32),
        scratch_types=[
            pltpu.VMEM((NCHUNK, CHUNK), jnp.int32),
            pltpu.VMEM((CHUNK, DIM), jnp.float32),
            pltpu.VMEM((CHUNK, DIM), jnp.float32),
            pltpu.VMEM((CHUNK, DIM), jnp.float32),
            pltpu.VMEM((CHUNK, DIM), jnp.float32),
            pltpu.VMEM((CHUNK, ROW), jnp.float32),
            pltpu.SemaphoreType.DMA,
        ],
    )(_body)
    out = f(idx, scalar, vector_i, vector_j, vector_k)
    return out.reshape(BATCH, SEQ, DIM, 4)


# speedup vs baseline: 1.6223x; 1.6223x over previous
import functools
import math

import jax
import jax.numpy as jnp
from jax import lax
from jax.experimental import pallas as pl
from jax.experimental.pallas import tpu as pltpu
from jax.experimental.pallas import tpu_sc as plsc

VOCAB = 1000000
DIM = 64
BATCH = 1024
SEQ = 200
NW = 32
BBLK = BATCH // 128
NFS = 4
UNITS = SEQ * NFS
PER_W = UNITS // NW
NCH = BATCH // 128
SKEW = 65

_LN10000_OVER_DIM = math.log(10000.0) / DIM


def _rsqrt_nr(s):
    half = s * 0.5
    bits = lax.bitcast_convert_type(s, jnp.int32)
    bits = jnp.int32(0x5F3759DF) - (bits >> 1)
    y = lax.bitcast_convert_type(bits, jnp.float32)
    y = y * (1.5 - half * y * y)
    y = y * (1.5 - half * y * y)
    return y


def _body(idx_hbm, scalar_hbm, vi_hbm, vj_hbm, vk_hbm, out_hbm,
          idx_v, gidx, gr, gi, gj, gk, stag, obuf, scv, sem, osem):
    wid = lax.axis_index("s") * 2 + lax.axis_index("c")
    ubase = wid * PER_W

    lane = lax.iota(jnp.int32, 16)
    lane_skew = lane * SKEW
    lane_f = lane.astype(jnp.float32)
    for g in range(NFS):
        scv[g, :] = jnp.exp((lane_f + (g * 16)) * (-_LN10000_OVER_DIM))

    def prep(ch, fs):
        m = ch % 2
        for i in range(8):
            v = idx_v[pl.ds(ch * 128 + i * 16, 16)]
            gidx[m, pl.ds(i * 16, 16)] = v * 4 + fs
        irow = gidx.at[m]
        pltpu.async_copy(scalar_hbm.at[irow], gr.at[m], sem)
        pltpu.async_copy(vi_hbm.at[irow], gi.at[m], sem)
        pltpu.async_copy(vj_hbm.at[irow], gj.at[m], sem)
        pltpu.async_copy(vk_hbm.at[irow], gk.at[m], sem)

    def wait_gathers(ch):
        m = ch % 2
        irow = gidx.at[m]
        pltpu.make_async_copy(scalar_hbm.at[irow], gr.at[m], sem).wait()
        pltpu.make_async_copy(vi_hbm.at[irow], gi.at[m], sem).wait()
        pltpu.make_async_copy(vj_hbm.at[irow], gj.at[m], sem).wait()
        pltpu.make_async_copy(vk_hbm.at[irow], gk.at[m], sem).wait()

    def out_dma_desc(u):
        s = u // NFS
        fs = u % NFS
        return pltpu.make_async_copy(obuf, out_hbm.at[s, fs], osem)

    def compute_chunk(ch, scale):
        m = ch % 2
        grm, gim, gjm, gkm = gr.at[m], gi.at[m], gj.at[m], gk.at[m]

        @plsc.parallel_loop(0, 128, unroll=2)
        def n_body(b):
            r = grm[b, :]
            i = gim[b, :] * scale
            j = gjm[b, :] * scale
            k = gkm[b, :] * scale
            y = _rsqrt_nr(r * r + i * i + j * j + k * k + 1e-6)
            a0 = b * SKEW
            stag[pl.ds(a0, 16)] = r * y
            stag[pl.ds(a0 + 16, 16)] = i * y
            stag[pl.ds(a0 + 32, 16)] = j * y
            stag[pl.ds(a0 + 48, 16)] = k * y

    def repack_chunk(ch):
        cb = ch * 512

        @plsc.parallel_loop(0, BBLK)
        def j_body(jj):
            src0 = jj * (16 * SKEW)
            dst0 = cb + jj * 16
            for f in range(16):
                for cc in range(4):
                    v = plsc.load_gather(
                        stag, [lane_skew + (src0 + cc * 16 + f)])
                    obuf[pl.ds(dst0 + f * 4096 + cc * 128, 16)] = v

    def unit_body(t, carry):
        u = ubase + t
        s = u // NFS
        fs = u % NFS
        pltpu.sync_copy(idx_hbm.at[s], idx_v)
        scale = scv[fs, :]
        prep(0, fs)

        @pl.when(t >= 1)
        def _():
            out_dma_desc(u - 1).wait()

        def chunk_body(ch, carry2):
            @pl.when(ch < NCH - 1)
            def _():
                prep(ch + 1, fs)

            wait_gathers(ch)
            compute_chunk(ch, scale)
            repack_chunk(ch)
            return carry2

        lax.fori_loop(0, NCH, chunk_body, 0)
        out_dma_desc(u).start()
        return carry

    lax.fori_loop(0, PER_W, unit_body, 0)
    out_dma_desc(ubase + PER_W - 1).wait()


def kernel(x, scalar, vector_i, vector_j, vector_k):
    idx = x.T
    tabs = [t.reshape(4 * VOCAB, 16)
            for t in (scalar, vector_i, vector_j, vector_k)]
    mesh = plsc.VectorSubcoreMesh(core_axis_name="c", subcore_axis_name="s")
    f = functools.partial(
        pl.kernel,
        mesh=mesh,
        compiler_params=pltpu.CompilerParams(use_tc_tiling_on_sc=False,
                                             needs_layout_passes=False),
        out_type=jax.ShapeDtypeStruct((SEQ, NFS, 16 * BBLK * 4 * 128),
                                      jnp.float32),
        scratch_types=[
            pltpu.VMEM((BATCH,), jnp.int32),
            pltpu.VMEM((2, 128), jnp.int32),
            pltpu.VMEM((2, 128, 16), jnp.float32),
            pltpu.VMEM((2, 128, 16), jnp.float32),
            pltpu.VMEM((2, 128, 16), jnp.float32),
            pltpu.VMEM((2, 128, 16), jnp.float32),
            pltpu.VMEM((128 * SKEW,), jnp.float32),
            pltpu.VMEM((16 * BBLK * 4 * 128,), jnp.float32),
            pltpu.VMEM((NFS, 16), jnp.float32),
            pltpu.SemaphoreType.DMA,
            pltpu.SemaphoreType.DMA,
        ],
    )(_body)
    out = f(idx, *tabs)
    out5 = out.reshape(SEQ, DIM, BBLK, 4, 128)
    return out5.transpose(2, 4, 0, 1, 3).reshape(BATCH, SEQ, DIM, 4)
